# Initial kernel scaffold; baseline (speedup 1.0000x reference)
import jax
import jax.numpy as jnp
from jax.experimental import pallas as pl


def kernel(x, edge_index, W1, b1, W2, b2, W3, b3):
    raise NotImplementedError("write your pallas kernel here")


# speedup vs baseline: 5.1461x; 5.1461x over previous
import functools

import jax
import jax.numpy as jnp
from jax import lax
from jax.experimental import pallas as pl
from jax.experimental.pallas import tpu as pltpu
from jax.experimental.pallas import tpu_sc as plsc

N = 10000
NP = 10240
D = 128
E = 320000
EP = 322560
NC = 2
NS = 16
NW = NC * NS
EPW = EP // NW
CH = 112
NCHUNK = EPW // CH
NPAIRW = NCHUNK // 2
RPS = NP // NS
SR = 80


def _make_segsum():
    mesh = plsc.VectorSubcoreMesh(core_axis_name="c", subcore_axis_name="s",
                                  num_cores=2)

    @functools.partial(
        pl.kernel,
        out_type=jax.ShapeDtypeStruct((NC, NP, D), jnp.float32),
        mesh=mesh,
        scratch_types=[
            pltpu.VMEM((4, CH), jnp.int32),
            pltpu.VMEM((2, CH, D), jnp.float32),
            pltpu.VMEM((SR, D), jnp.float32),
            pltpu.SemaphoreType.DMA,
            pltpu.SemaphoreType.DMA,
            pltpu.VMEM_SHARED((NP, D), jnp.float32),
        ],
    )
    def seg(h_hbm, pk_hbm, zeros_hbm, out_hbm,
            idxb, rows, zbuf, sem_a, sem_b, acc):
        c = lax.axis_index("c")
        s = lax.axis_index("s")
        row0 = s * RPS
        pltpu.sync_copy(zeros_hbm.at[pl.ds(0, SR)], zbuf)
        for k in range(RPS // SR):
            pltpu.sync_copy(zbuf, acc.at[pl.ds(row0 + k * SR, SR)])
        plsc.subcore_barrier()
        pair0 = (c * NS + s) * NPAIRW

        def pair(p, carry):
            pltpu.sync_copy(pk_hbm.at[pair0 + p], idxb)
            ga = pltpu.async_copy(h_hbm.at[idxb.at[0]], rows.at[0], sem_a)
            gb = pltpu.async_copy(h_hbm.at[idxb.at[1]], rows.at[1], sem_b)
            ga.wait()
            pltpu.sync_copy(rows.at[0], acc.at[idxb.at[2]], add=True)
            gb.wait()
            pltpu.sync_copy(rows.at[1], acc.at[idxb.at[3]], add=True)
            return carry

        lax.fori_loop(0, NPAIRW, pair, 0)
        plsc.subcore_barrier()
        for k in range(RPS // SR):
            pltpu.sync_copy(acc.at[pl.ds(row0 + k * SR, SR)], zbuf)
            pltpu.sync_copy(zbuf, out_hbm.at[c, pl.ds(row0 + k * SR, SR)])

    return seg


def _make_degcount():
    mesh = plsc.VectorSubcoreMesh(core_axis_name="c", subcore_axis_name="s",
                                  num_cores=2)

    @functools.partial(
        pl.kernel,
        out_type=jax.ShapeDtypeStruct((NC, NP, D), jnp.float32),
        mesh=mesh,
        scratch_types=[
            pltpu.VMEM((CH,), jnp.int32),
            pltpu.VMEM((CH, D), jnp.float32),
            pltpu.VMEM((SR, D), jnp.float32),
            pltpu.VMEM_SHARED((NP, D), jnp.float32),
        ],
    )
    def deg(dst_hbm, zeros_hbm, ones_hbm, deg_hbm, idx_d, ones_v, dbuf,
            dacc):
        c = lax.axis_index("c")
        s = lax.axis_index("s")
        row0 = s * RPS
        pltpu.sync_copy(zeros_hbm.at[pl.ds(0, SR)], dbuf)
        for k in range(RPS // SR):
            pltpu.sync_copy(dbuf, dacc.at[pl.ds(row0 + k * SR, SR)])
        pltpu.sync_copy(ones_hbm, ones_v)
        plsc.subcore_barrier()
        base_w = (c * NS + s) * EPW

        def body(j, carry):
            base = base_w + j * CH
            pltpu.sync_copy(dst_hbm.at[pl.ds(base, CH)], idx_d)
            pltpu.sync_copy(ones_v, dacc.at[idx_d], add=True)
            return carry

        lax.fori_loop(0, NCHUNK, body, 0)
        plsc.subcore_barrier()
        for k in range(RPS // SR):
            pltpu.sync_copy(dacc.at[pl.ds(row0 + k * SR, SR)], dbuf)
            pltpu.sync_copy(dbuf, deg_hbm.at[c, pl.ds(row0 + k * SR, SR)])

    return deg


_segsum = _make_segsum()
_degcount = _make_degcount()

BR = 1280


def _tc_layer(h, S, dg, W, b2, flag):
    Wt = W[:D]
    Wb = W[D:]

    def body(h_ref, s_ref, d_ref, wt_ref, wb_ref, b_ref, f_ref, o_ref):
        deg = jnp.maximum(d_ref[...], 1.0)
        agg = (s_ref[0] + s_ref[1]) / deg
        acc = jnp.dot(h_ref[...], wt_ref[...],
                      preferred_element_type=jnp.float32)
        acc = acc + jnp.dot(agg, wb_ref[...],
                            preferred_element_type=jnp.float32)
        acc = acc + b_ref[...]
        acc = jnp.where(f_ref[...] > 0.0, jnp.maximum(acc, 0.0), acc)
        o_ref[...] = acc

    return pl.pallas_call(
        body,
        grid=(NP // BR,),
        in_specs=[
            pl.BlockSpec((BR, D), lambda i: (i, 0)),
            pl.BlockSpec((NC, BR, D), lambda i: (0, i, 0)),
            pl.BlockSpec((BR, 1), lambda i: (i, 0)),
            pl.BlockSpec((D, D), lambda i: (0, 0)),
            pl.BlockSpec((D, D), lambda i: (0, 0)),
            pl.BlockSpec((1, D), lambda i: (0, 0)),
            pl.BlockSpec((1, D), lambda i: (0, 0)),
        ],
        out_specs=pl.BlockSpec((BR, D), lambda i: (i, 0)),
        out_shape=jax.ShapeDtypeStruct((NP, D), jnp.float32),
    )(h, S, dg, Wt, Wb, b2, flag)


def kernel(x, edge_index, W1, b1, W2, b2, W3, b3):
    src = edge_index[0].astype(jnp.int32)
    dst = edge_index[1].astype(jnp.int32)
    src = jnp.concatenate([src, jnp.zeros((EP - E,), jnp.int32)])
    dst = jnp.concatenate([dst, jnp.full((EP - E,), NP - 1, jnp.int32)])
    sp = src.reshape(-1, 2, CH)
    dp = dst.reshape(-1, 2, CH)
    packed = jnp.concatenate([sp, dp], axis=1)
    x = jnp.pad(x, ((0, NP - N), (0, 0)))
    zeros = jnp.zeros((SR, D), jnp.float32)
    ones = jnp.ones((CH, D), jnp.float32)
    Dg = _degcount(dst, zeros, ones)
    dg = Dg[0, :, 0:1] + Dg[1, :, 0:1]
    Ws = jnp.stack([W1, W2, W3])
    bs = jnp.stack([b1.reshape(1, D), b2.reshape(1, D), b3.reshape(1, D)])
    flags = jnp.ones((3, 1, D), jnp.float32).at[2].set(0.0)

    def step(h, wbf):
        W, b, f = wbf
        S = _segsum(h, packed, zeros)
        return _tc_layer(h, S, dg, W, b, f), None

    hf, _ = lax.scan(step, x, (Ws, bs, flags))
    return hf[:N]
